# Initial kernel scaffold; baseline (speedup 1.0000x reference)
import jax
import jax.numpy as jnp
from jax.experimental import pallas as pl


def kernel(nodes, edge_index, edge_attr, W1, b1, W2, b2, W3, b3, root1, bias1, root2, bias2):
    raise NotImplementedError("write your pallas kernel here")


# speedup vs baseline: 1.2909x; 1.2909x over previous
import functools

import jax
import jax.numpy as jnp
from jax import lax
from jax.experimental import pallas as pl
from jax.experimental.pallas import tpu as pltpu
from jax.experimental.pallas import tpu_sc as plsc

_N = 10000
_E = 320000
_D = 16
_ED = 16
_KD = 64

_NC = 2
_NS = 16
_NW = _NC * _NS
_PER_W = _E // _NW
_CH = 128
_NFULL = _PER_W // _CH
_TAIL = _PER_W - _NFULL * _CH
_ZR = _N // _NS

_TE = 2560

_INV_SQRT2 = 0.7071067811865476


def _gelu_exact(x):
    return 0.5 * x * (1.0 + lax.erf(x * _INV_SQRT2))


def _sc_gather(table, idx):
    mesh = plsc.VectorSubcoreMesh(core_axis_name="c", subcore_axis_name="s")

    @functools.partial(
        pl.kernel,
        mesh=mesh,
        out_type=jax.ShapeDtypeStruct((_E, _D), jnp.float32),
        scratch_types=[
            pltpu.VMEM((_CH,), jnp.int32),
            pltpu.VMEM((_CH, _D), jnp.float32),
            pltpu.VMEM((_TAIL,), jnp.int32),
            pltpu.VMEM((_TAIL, _D), jnp.float32),
            pltpu.SemaphoreType.DMA,
        ],
        compiler_params=pltpu.CompilerParams(use_tc_tiling_on_sc=False),
    )
    def k(table_hbm, idx_hbm, out_hbm, idx_v, rows_v, idx_t, rows_t, sem):
        wid = lax.axis_index("s") * _NC + lax.axis_index("c")
        base_w = wid * _PER_W

        def body(j, carry):
            base = base_w + j * _CH
            pltpu.sync_copy(idx_hbm.at[pl.ds(base, _CH)], idx_v)
            pltpu.async_copy(table_hbm.at[idx_v], rows_v, sem).wait()
            pltpu.sync_copy(rows_v, out_hbm.at[pl.ds(base, _CH)])
            return carry

        lax.fori_loop(0, _NFULL, body, 0)
        base = base_w + _NFULL * _CH
        pltpu.sync_copy(idx_hbm.at[pl.ds(base, _TAIL)], idx_t)
        pltpu.async_copy(table_hbm.at[idx_t], rows_t, sem).wait()
        pltpu.sync_copy(rows_t, out_hbm.at[pl.ds(base, _TAIL)])

    return k(table, idx)


def _sc_scatter(msg, dst):
    mesh = plsc.VectorSubcoreMesh(core_axis_name="c", subcore_axis_name="s")

    @functools.partial(
        pl.kernel,
        mesh=mesh,
        out_type=jax.ShapeDtypeStruct((_NC, _N, _D), jnp.float32),
        scratch_types=[
            pltpu.VMEM((_CH,), jnp.int32),
            pltpu.VMEM((_CH, _D), jnp.float32),
            pltpu.VMEM((_TAIL,), jnp.int32),
            pltpu.VMEM((_TAIL, _D), jnp.float32),
            pltpu.VMEM((_ZR, _D), jnp.float32),
            pltpu.VMEM_SHARED((_N, _D), jnp.float32),
            pltpu.SemaphoreType.DMA,
        ],
        compiler_params=pltpu.CompilerParams(use_tc_tiling_on_sc=False),
    )
    def k(msg_hbm, dst_hbm, out_hbm, idx_v, msg_v, idx_t, msg_t, z_v, acc, sem):
        cid = lax.axis_index("c")
        sid = lax.axis_index("s")
        wid = sid * _NC + cid
        zero = jnp.zeros((_D,), jnp.float32)

        def zb(i, carry):
            z_v[i, :] = zero
            return carry

        lax.fori_loop(0, _ZR, zb, 0)
        pltpu.sync_copy(z_v, acc.at[pl.ds(sid * _ZR, _ZR)])
        plsc.subcore_barrier()

        base_w = wid * _PER_W

        def body(j, carry):
            base = base_w + j * _CH
            pltpu.sync_copy(dst_hbm.at[pl.ds(base, _CH)], idx_v)
            pltpu.sync_copy(msg_hbm.at[pl.ds(base, _CH)], msg_v)
            pltpu.sync_copy(msg_v, acc.at[idx_v], add=True)
            return carry

        lax.fori_loop(0, _NFULL, body, 0)
        base = base_w + _NFULL * _CH
        pltpu.sync_copy(dst_hbm.at[pl.ds(base, _TAIL)], idx_t)
        pltpu.sync_copy(msg_hbm.at[pl.ds(base, _TAIL)], msg_t)
        pltpu.sync_copy(msg_t, acc.at[idx_t], add=True)

        plsc.subcore_barrier()
        pltpu.sync_copy(acc.at[pl.ds(sid * _ZR, _ZR)],
                        out_hbm.at[cid, pl.ds(sid * _ZR, _ZR)])

    return k(msg, dst)


def _tc_msg(ea, xj, W1, b1, W2, b2, W3, b3):
    def body(ea_ref, xj_ref, w1, b1r, w2, b2r, w3, b3r, out_ref):
        h = jnp.dot(ea_ref[...], w1[...], preferred_element_type=jnp.float32)
        h = _gelu_exact(h + b1r[...])
        h = jnp.dot(h, w2[...], preferred_element_type=jnp.float32)
        h = _gelu_exact(h + b2r[...])
        w = jnp.dot(h, w3[...], preferred_element_type=jnp.float32) + b3r[...]
        xjv = xj_ref[...]
        acc = xjv[:, 0:1] * w[:, 0:_D]
        for i in range(1, _D):
            acc = acc + xjv[:, i:i + 1] * w[:, i * _D:(i + 1) * _D]
        out_ref[...] = acc

    return pl.pallas_call(
        body,
        grid=(_E // _TE,),
        in_specs=[
            pl.BlockSpec((_TE, _ED), lambda i: (i, 0)),
            pl.BlockSpec((_TE, _D), lambda i: (i, 0)),
            pl.BlockSpec((_ED, _KD), lambda i: (0, 0)),
            pl.BlockSpec((1, _KD), lambda i: (0, 0)),
            pl.BlockSpec((_KD, _KD), lambda i: (0, 0)),
            pl.BlockSpec((1, _KD), lambda i: (0, 0)),
            pl.BlockSpec((_KD, _D * _D), lambda i: (0, 0)),
            pl.BlockSpec((1, _D * _D), lambda i: (0, 0)),
        ],
        out_specs=pl.BlockSpec((_TE, _D), lambda i: (i, 0)),
        out_shape=jax.ShapeDtypeStruct((_E, _D), jnp.float32),
    )(ea, xj, W1, b1, W2, b2, W3, b3)


def _tc_final(x, parts, root, bias, use_gelu):
    def body(x_ref, p_ref, r_ref, b_ref, out_ref):
        out = p_ref[0] + p_ref[1] + b_ref[...]
        out = out + jnp.dot(x_ref[...], r_ref[...],
                            preferred_element_type=jnp.float32)
        if use_gelu:
            out = _gelu_exact(out)
        out_ref[...] = out

    return pl.pallas_call(
        body,
        out_shape=jax.ShapeDtypeStruct((_N, _D), jnp.float32),
    )(x, parts, root, bias)


def kernel(nodes, edge_index, edge_attr, W1, b1, W2, b2, W3, b3,
           root1, bias1, root2, bias2):
    src = edge_index[0].astype(jnp.int32)
    dst = edge_index[1].astype(jnp.int32)
    b1r = b1.reshape(1, _KD)
    b2r = b2.reshape(1, _KD)
    b3r = b3.reshape(1, _D * _D)
    bias1r = bias1.reshape(1, _D)
    bias2r = bias2.reshape(1, _D)

    def gno_pass(x, root, biasr, use_gelu):
        xj = _sc_gather(x, src)
        msg = _tc_msg(edge_attr, xj, W1, b1r, W2, b2r, W3, b3r)
        parts = _sc_scatter(msg, dst)
        return _tc_final(x, parts, root, biasr, use_gelu)

    h = gno_pass(nodes, root1, bias1r, True)
    return gno_pass(h, root2, bias2r, False)


# speedup vs baseline: 4.1103x; 3.1841x over previous
import functools

import jax
import jax.numpy as jnp
from jax import lax
from jax.experimental import pallas as pl
from jax.experimental.pallas import tpu as pltpu
from jax.experimental.pallas import tpu_sc as plsc

_N = 10000
_E = 320000
_D = 16
_ED = 16
_KD = 64

_NC = 2
_NS = 16
_NW = _NC * _NS
_CH = 128
_NCK = _E // _CH
_CPW = _NCK // _NW
_XTRA = _NCK - _CPW * _NW
_SUP = 6
_NSUP = _CPW // _SUP
_SROWS = _SUP * _CH
_ZR = _N // _NS

_TE = 2560

_INV_SQRT2 = 0.7071067811865476


def _gelu_exact(x):
    return 0.5 * x * (1.0 + lax.erf(x * _INV_SQRT2))


def _sc_gather(table, idx2d):
    mesh = plsc.VectorSubcoreMesh(core_axis_name="c", subcore_axis_name="s")

    @functools.partial(
        pl.kernel,
        mesh=mesh,
        out_type=jax.ShapeDtypeStruct((_E, _D), jnp.float32),
        scratch_types=[
            pltpu.VMEM((_CPW + 1, _CH), jnp.int32),
            pltpu.VMEM((2, _SROWS, _D), jnp.float32),
            pltpu.VMEM((_CH, _D), jnp.float32),
            pltpu.SemaphoreType.DMA,
            pltpu.SemaphoreType.DMA,
        ],
        compiler_params=pltpu.CompilerParams(use_tc_tiling_on_sc=False),
    )
    def k(table_hbm, idx_hbm, out_hbm, idxbuf, rows2, rows_x, gsem, wsem):
        wid = lax.axis_index("s") * _NC + lax.axis_index("c")
        row0 = wid * _CPW
        pltpu.sync_copy(idx_hbm.at[pl.ds(row0, _CPW)],
                        idxbuf.at[pl.ds(0, _CPW)])

        def super_body(j, carry):
            p = lax.rem(j, 2)

            @pl.when(j >= 2)
            def _():
                pltpu.make_async_copy(
                    out_hbm.at[pl.ds(0, _SROWS)], rows2.at[p], wsem).wait()

            handles = []
            for b in range(_SUP):
                handles.append(pltpu.async_copy(
                    table_hbm.at[idxbuf.at[j * _SUP + b]],
                    rows2.at[p, pl.ds(b * _CH, _CH)], gsem))
            for h in handles:
                h.wait()
            pltpu.async_copy(
                rows2.at[p],
                out_hbm.at[pl.ds((row0 + j * _SUP) * _CH, _SROWS)], wsem)
            return carry

        lax.fori_loop(0, _NSUP, super_body, 0)
        pltpu.make_async_copy(
            out_hbm.at[pl.ds(0, _SROWS)], rows2.at[0], wsem).wait()
        pltpu.make_async_copy(
            out_hbm.at[pl.ds(0, _SROWS)], rows2.at[1], wsem).wait()

        @pl.when(wid < _XTRA)
        def _():
            xrow = _NW * _CPW + wid
            pltpu.sync_copy(idx_hbm.at[xrow], idxbuf.at[_CPW])
            pltpu.async_copy(
                table_hbm.at[idxbuf.at[_CPW]], rows_x, gsem).wait()
            pltpu.sync_copy(rows_x, out_hbm.at[pl.ds(xrow * _CH, _CH)])

    return k(table, idx2d)


def _sc_scatter(msg, dst2d):
    mesh = plsc.VectorSubcoreMesh(core_axis_name="c", subcore_axis_name="s")

    @functools.partial(
        pl.kernel,
        mesh=mesh,
        out_type=jax.ShapeDtypeStruct((_NC, _N, _D), jnp.float32),
        scratch_types=[
            pltpu.VMEM((_CPW + 1, _CH), jnp.int32),
            pltpu.VMEM((2, _SROWS, _D), jnp.float32),
            pltpu.VMEM((_CH, _D), jnp.float32),
            pltpu.VMEM((_ZR, _D), jnp.float32),
            pltpu.VMEM_SHARED((_N, _D), jnp.float32),
            pltpu.SemaphoreType.DMA,
        ],
        compiler_params=pltpu.CompilerParams(use_tc_tiling_on_sc=False),
    )
    def k(msg_hbm, dst_hbm, out_hbm, idxbuf, msg2, msg_x, z_v, acc, lsem):
        cid = lax.axis_index("c")
        sid = lax.axis_index("s")
        wid = sid * _NC + cid
        row0 = wid * _CPW
        zero = jnp.zeros((_D,), jnp.float32)

        def zb(i, carry):
            z_v[i, :] = zero
            return carry

        lax.fori_loop(0, _ZR, zb, 0)
        pltpu.sync_copy(z_v, acc.at[pl.ds(sid * _ZR, _ZR)])
        pltpu.sync_copy(dst_hbm.at[pl.ds(row0, _CPW)],
                        idxbuf.at[pl.ds(0, _CPW)])
        plsc.subcore_barrier()

        pltpu.async_copy(msg_hbm.at[pl.ds(row0 * _CH, _SROWS)],
                         msg2.at[0], lsem)

        def super_body(j, carry):
            p = lax.rem(j, 2)
            pltpu.make_async_copy(
                msg_hbm.at[pl.ds(0, _SROWS)], msg2.at[p], lsem).wait()

            @pl.when(j < _NSUP - 1)
            def _():
                pltpu.async_copy(
                    msg_hbm.at[pl.ds((row0 + (j + 1) * _SUP) * _CH, _SROWS)],
                    msg2.at[1 - p], lsem)

            for b in range(_SUP):
                pltpu.sync_copy(msg2.at[p, pl.ds(b * _CH, _CH)],
                                acc.at[idxbuf.at[j * _SUP + b]], add=True)
            return carry

        lax.fori_loop(0, _NSUP, super_body, 0)

        @pl.when(wid < _XTRA)
        def _():
            xrow = _NW * _CPW + wid
            pltpu.sync_copy(dst_hbm.at[xrow], idxbuf.at[_CPW])
            pltpu.sync_copy(msg_hbm.at[pl.ds(xrow * _CH, _CH)], msg_x)
            pltpu.sync_copy(msg_x, acc.at[idxbuf.at[_CPW]], add=True)

        plsc.subcore_barrier()
        pltpu.sync_copy(acc.at[pl.ds(sid * _ZR, _ZR)],
                        out_hbm.at[cid, pl.ds(sid * _ZR, _ZR)])

    return k(msg, dst2d)


def _tc_msg(ea, xj, W1, b1, W2, b2, W3, b3):
    def body(ea_ref, xj_ref, w1, b1r, w2, b2r, w3, b3r, out_ref):
        h = jnp.dot(ea_ref[...], w1[...], preferred_element_type=jnp.float32)
        h = _gelu_exact(h + b1r[...])
        h = jnp.dot(h, w2[...], preferred_element_type=jnp.float32)
        h = _gelu_exact(h + b2r[...])
        w = jnp.dot(h, w3[...], preferred_element_type=jnp.float32) + b3r[...]
        ii = lax.broadcasted_iota(jnp.int32, (_D, _D * _D), 0)
        ll = lax.broadcasted_iota(jnp.int32, (_D, _D * _D), 1)
        rmat = (ll // _D == ii).astype(jnp.float32)
        lo = lax.broadcasted_iota(jnp.int32, (_D * _D, _D), 0)
        oo = lax.broadcasted_iota(jnp.int32, (_D * _D, _D), 1)
        smat = (lo % _D == oo).astype(jnp.float32)
        xrep = jnp.dot(xj_ref[...], rmat, preferred_element_type=jnp.float32)
        out_ref[...] = jnp.dot(xrep * w, smat,
                               preferred_element_type=jnp.float32)

    return pl.pallas_call(
        body,
        grid=(_E // _TE,),
        in_specs=[
            pl.BlockSpec((_TE, _ED), lambda i: (i, 0)),
            pl.BlockSpec((_TE, _D), lambda i: (i, 0)),
            pl.BlockSpec((_ED, _KD), lambda i: (0, 0)),
            pl.BlockSpec((1, _KD), lambda i: (0, 0)),
            pl.BlockSpec((_KD, _KD), lambda i: (0, 0)),
            pl.BlockSpec((1, _KD), lambda i: (0, 0)),
            pl.BlockSpec((_KD, _D * _D), lambda i: (0, 0)),
            pl.BlockSpec((1, _D * _D), lambda i: (0, 0)),
        ],
        out_specs=pl.BlockSpec((_TE, _D), lambda i: (i, 0)),
        out_shape=jax.ShapeDtypeStruct((_E, _D), jnp.float32),
    )(ea, xj, W1, b1, W2, b2, W3, b3)


def _tc_final(x, parts, root, bias, use_gelu):
    def body(x_ref, p_ref, r_ref, b_ref, out_ref):
        out = p_ref[0] + p_ref[1] + b_ref[...]
        out = out + jnp.dot(x_ref[...], r_ref[...],
                            preferred_element_type=jnp.float32)
        if use_gelu:
            out = _gelu_exact(out)
        out_ref[...] = out

    return pl.pallas_call(
        body,
        out_shape=jax.ShapeDtypeStruct((_N, _D), jnp.float32),
    )(x, parts, root, bias)


def kernel(nodes, edge_index, edge_attr, W1, b1, W2, b2, W3, b3,
           root1, bias1, root2, bias2):
    src2d = edge_index[0].astype(jnp.int32).reshape(_NCK, _CH)
    dst2d = edge_index[1].astype(jnp.int32).reshape(_NCK, _CH)
    b1r = b1.reshape(1, _KD)
    b2r = b2.reshape(1, _KD)
    b3r = b3.reshape(1, _D * _D)
    bias1r = bias1.reshape(1, _D)
    bias2r = bias2.reshape(1, _D)

    def gno_pass(x, root, biasr, use_gelu):
        xj = _sc_gather(x, src2d)
        msg = _tc_msg(edge_attr, xj, W1, b1r, W2, b2r, W3, b3r)
        parts = _sc_scatter(msg, dst2d)
        return _tc_final(x, parts, root, biasr, use_gelu)

    h = gno_pass(nodes, root1, bias1r, True)
    return gno_pass(h, root2, bias2r, False)


# speedup vs baseline: 5.6716x; 1.3799x over previous
import functools

import jax
import jax.numpy as jnp
from jax import lax
from jax.experimental import pallas as pl
from jax.experimental.pallas import tpu as pltpu
from jax.experimental.pallas import tpu_sc as plsc

_N = 10000
_E = 320000
_D = 16
_ED = 16
_KD = 64

_NC = 2
_NS = 16
_NW = _NC * _NS
_CH = 128
_NCK = _E // _CH
_CPW = _NCK // _NW
_XTRA = _NCK - _CPW * _NW
_SUP = 6
_NSUP = _CPW // _SUP
_SROWS = _SUP * _CH
_ZR = _N // _NS

_TE = 2560

_INV_SQRT2 = 0.7071067811865476


def _gelu_exact(x):
    return 0.5 * x * (1.0 + lax.erf(x * _INV_SQRT2))


def _sc_gather(table, idx2d):
    mesh = plsc.VectorSubcoreMesh(core_axis_name="c", subcore_axis_name="s")

    @functools.partial(
        pl.kernel,
        mesh=mesh,
        out_type=jax.ShapeDtypeStruct((_E, _D), jnp.float32),
        scratch_types=[
            pltpu.VMEM((_CPW + 1, _CH), jnp.int32),
            pltpu.VMEM((2, _SROWS, _D), jnp.float32),
            pltpu.VMEM((_CH, _D), jnp.float32),
            pltpu.SemaphoreType.DMA,
            pltpu.SemaphoreType.DMA,
        ],
        compiler_params=pltpu.CompilerParams(use_tc_tiling_on_sc=False),
    )
    def k(table_hbm, idx_hbm, out_hbm, idxbuf, rows2, rows_x, gsem, wsem):
        wid = lax.axis_index("s") * _NC + lax.axis_index("c")
        row0 = wid * _CPW
        pltpu.sync_copy(idx_hbm.at[pl.ds(row0, _CPW)],
                        idxbuf.at[pl.ds(0, _CPW)])

        def super_body(j, carry):
            p = lax.rem(j, 2)

            @pl.when(j >= 2)
            def _():
                pltpu.make_async_copy(
                    out_hbm.at[pl.ds(0, _SROWS)], rows2.at[p], wsem).wait()

            handles = []
            for b in range(_SUP):
                handles.append(pltpu.async_copy(
                    table_hbm.at[idxbuf.at[j * _SUP + b]],
                    rows2.at[p, pl.ds(b * _CH, _CH)], gsem))
            for h in handles:
                h.wait()
            pltpu.async_copy(
                rows2.at[p],
                out_hbm.at[pl.ds((row0 + j * _SUP) * _CH, _SROWS)], wsem)
            return carry

        lax.fori_loop(0, _NSUP, super_body, 0)
        pltpu.make_async_copy(
            out_hbm.at[pl.ds(0, _SROWS)], rows2.at[0], wsem).wait()
        pltpu.make_async_copy(
            out_hbm.at[pl.ds(0, _SROWS)], rows2.at[1], wsem).wait()

        @pl.when(wid < _XTRA)
        def _():
            xrow = _NW * _CPW + wid
            pltpu.sync_copy(idx_hbm.at[xrow], idxbuf.at[_CPW])
            pltpu.async_copy(
                table_hbm.at[idxbuf.at[_CPW]], rows_x, gsem).wait()
            pltpu.sync_copy(rows_x, out_hbm.at[pl.ds(xrow * _CH, _CH)])

    return k(table, idx2d)


def _sc_scatter(msg, dst2d):
    mesh = plsc.VectorSubcoreMesh(core_axis_name="c", subcore_axis_name="s")

    @functools.partial(
        pl.kernel,
        mesh=mesh,
        out_type=jax.ShapeDtypeStruct((_NC, _N, _D), jnp.float32),
        scratch_types=[
            pltpu.VMEM((_CPW + 1, _CH), jnp.int32),
            pltpu.VMEM((2, _SROWS, _D), jnp.float32),
            pltpu.VMEM((_CH, _D), jnp.float32),
            pltpu.VMEM((_ZR, _D), jnp.float32),
            pltpu.VMEM_SHARED((_N, _D), jnp.float32),
            pltpu.SemaphoreType.DMA,
        ],
        compiler_params=pltpu.CompilerParams(use_tc_tiling_on_sc=False),
    )
    def k(msg_hbm, dst_hbm, out_hbm, idxbuf, msg2, msg_x, z_v, acc, lsem):
        cid = lax.axis_index("c")
        sid = lax.axis_index("s")
        wid = sid * _NC + cid
        row0 = wid * _CPW
        zero = jnp.zeros((_D,), jnp.float32)

        def zb(i, carry):
            z_v[i, :] = zero
            return carry

        lax.fori_loop(0, _ZR, zb, 0)
        pltpu.sync_copy(z_v, acc.at[pl.ds(sid * _ZR, _ZR)])
        pltpu.sync_copy(dst_hbm.at[pl.ds(row0, _CPW)],
                        idxbuf.at[pl.ds(0, _CPW)])
        plsc.subcore_barrier()

        pltpu.async_copy(msg_hbm.at[pl.ds(row0 * _CH, _SROWS)],
                         msg2.at[0], lsem)

        def super_body(j, carry):
            p = lax.rem(j, 2)
            pltpu.make_async_copy(
                msg_hbm.at[pl.ds(0, _SROWS)], msg2.at[p], lsem).wait()

            @pl.when(j < _NSUP - 1)
            def _():
                pltpu.async_copy(
                    msg_hbm.at[pl.ds((row0 + (j + 1) * _SUP) * _CH, _SROWS)],
                    msg2.at[1 - p], lsem)

            for b in range(_SUP):
                pltpu.sync_copy(msg2.at[p, pl.ds(b * _CH, _CH)],
                                acc.at[idxbuf.at[j * _SUP + b]], add=True)
            return carry

        lax.fori_loop(0, _NSUP, super_body, 0)

        @pl.when(wid < _XTRA)
        def _():
            xrow = _NW * _CPW + wid
            pltpu.sync_copy(dst_hbm.at[xrow], idxbuf.at[_CPW])
            pltpu.sync_copy(msg_hbm.at[pl.ds(xrow * _CH, _CH)], msg_x)
            pltpu.sync_copy(msg_x, acc.at[idxbuf.at[_CPW]], add=True)

        plsc.subcore_barrier()
        pltpu.sync_copy(acc.at[pl.ds(sid * _ZR, _ZR)],
                        out_hbm.at[cid, pl.ds(sid * _ZR, _ZR)])

    return k(msg, dst2d)


def _tc_msg(eap, xjp, W1p, b1p, W2p, b2p, W3p, b3p, Rp, Sp):
    _TR = _TE // 8

    def body(eap_ref, xjp_ref, w1p, b1r, w2p, b2r, w3p, b3r, rp, sp,
             out_ref):
        h = jnp.dot(eap_ref[...], w1p[...], preferred_element_type=jnp.float32)
        h = _gelu_exact(h + b1r[...])
        h = jnp.dot(h, w2p[...], preferred_element_type=jnp.float32)
        h = _gelu_exact(h + b2r[...])
        wp = jnp.dot(h, w3p[...], preferred_element_type=jnp.float32)
        wp = wp + b3r[...]
        xrep = jnp.dot(xjp_ref[...], rp[...],
                       preferred_element_type=jnp.float32)
        out_ref[...] = jnp.dot(xrep * wp, sp[...],
                               preferred_element_type=jnp.float32)

    return pl.pallas_call(
        body,
        grid=(_E // _TE,),
        in_specs=[
            pl.BlockSpec((_TR, 128), lambda i: (i, 0)),
            pl.BlockSpec((_TR, 128), lambda i: (i, 0)),
            pl.BlockSpec((128, 8 * _KD), lambda i: (0, 0)),
            pl.BlockSpec((1, 8 * _KD), lambda i: (0, 0)),
            pl.BlockSpec((8 * _KD, 8 * _KD), lambda i: (0, 0)),
            pl.BlockSpec((1, 8 * _KD), lambda i: (0, 0)),
            pl.BlockSpec((8 * _KD, 8 * _D * _D), lambda i: (0, 0)),
            pl.BlockSpec((1, 8 * _D * _D), lambda i: (0, 0)),
            pl.BlockSpec((128, 8 * _D * _D), lambda i: (0, 0)),
            pl.BlockSpec((8 * _D * _D, 128), lambda i: (0, 0)),
        ],
        out_specs=pl.BlockSpec((_TR, 128), lambda i: (i, 0)),
        out_shape=jax.ShapeDtypeStruct((_E * _D // 128, 128), jnp.float32),
    )(eap, xjp, W1p, b1p, W2p, b2p, W3p, b3p, Rp, Sp)


def _tc_final(x, parts, root, bias, use_gelu):
    def body(x_ref, p_ref, r_ref, b_ref, out_ref):
        out = p_ref[0] + p_ref[1] + b_ref[...]
        out = out + jnp.dot(x_ref[...], r_ref[...],
                            preferred_element_type=jnp.float32)
        if use_gelu:
            out = _gelu_exact(out)
        out_ref[...] = out

    return pl.pallas_call(
        body,
        out_shape=jax.ShapeDtypeStruct((_N, _D), jnp.float32),
    )(x, parts, root, bias)


def kernel(nodes, edge_index, edge_attr, W1, b1, W2, b2, W3, b3,
           root1, bias1, root2, bias2):
    src2d = edge_index[0].astype(jnp.int32).reshape(_NCK, _CH)
    dst2d = edge_index[1].astype(jnp.int32).reshape(_NCK, _CH)
    eye8 = jnp.eye(8, dtype=jnp.float32)
    W1p = jnp.kron(eye8, W1)
    W2p = jnp.kron(eye8, W2)
    W3p = jnp.kron(eye8, W3)
    b1p = jnp.tile(b1, 8).reshape(1, 8 * _KD)
    b2p = jnp.tile(b2, 8).reshape(1, 8 * _KD)
    b3p = jnp.tile(b3, 8).reshape(1, 8 * _D * _D)
    la = jnp.arange(128, dtype=jnp.int32)[:, None]
    lb = jnp.arange(8 * _D * _D, dtype=jnp.int32)[None, :]
    Rp = (la == _D * (lb // (_D * _D))
          + (lb % (_D * _D)) // _D).astype(jnp.float32)
    lc = jnp.arange(8 * _D * _D, dtype=jnp.int32)[:, None]
    ld = jnp.arange(128, dtype=jnp.int32)[None, :]
    Sp = ((lc // (_D * _D) == ld // _D)
          & (lc % _D == ld % _D)).astype(jnp.float32)
    eap = edge_attr.reshape(_E * _ED // 128, 128)
    bias1r = bias1.reshape(1, _D)
    bias2r = bias2.reshape(1, _D)

    def gno_pass(x, root, biasr, use_gelu):
        xj = _sc_gather(x, src2d)
        xjp = xj.reshape(_E * _D // 128, 128)
        msgp = _tc_msg(eap, xjp, W1p, b1p, W2p, b2p, W3p, b3p, Rp, Sp)
        parts = _sc_scatter(msgp.reshape(_E, _D), dst2d)
        return _tc_final(x, parts, root, biasr, use_gelu)

    h = gno_pass(nodes, root1, bias1r, True)
    return gno_pass(h, root2, bias2r, False)


# speedup vs baseline: 6.4695x; 1.1407x over previous
import functools

import jax
import jax.numpy as jnp
from jax import lax
from jax.experimental import pallas as pl
from jax.experimental.pallas import tpu as pltpu
from jax.experimental.pallas import tpu_sc as plsc

_N = 10000
_E = 320000
_D = 16
_ED = 16
_KD = 64

_NC = 2
_NS = 16
_NW = _NC * _NS
_CH = 128
_NCK = _E // _CH
_CPW = _NCK // _NW
_XTRA = _NCK - _CPW * _NW
_SUP = 6
_NSUP = _CPW // _SUP
_SROWS = _SUP * _CH
_ZR = _N // _NS

_TE = 2560

_INV_SQRT2 = 0.7071067811865476


def _gelu_exact(x):
    return 0.5 * x * (1.0 + lax.erf(x * _INV_SQRT2))


def _sc_gather(table, idx2d):
    mesh = plsc.VectorSubcoreMesh(core_axis_name="c", subcore_axis_name="s")

    @functools.partial(
        pl.kernel,
        mesh=mesh,
        out_type=jax.ShapeDtypeStruct((_E, _D), jnp.float32),
        scratch_types=[
            pltpu.VMEM((_CPW + 1, _CH), jnp.int32),
            pltpu.VMEM((2, _SROWS, _D), jnp.float32),
            pltpu.VMEM((_CH, _D), jnp.float32),
            pltpu.SemaphoreType.DMA,
            pltpu.SemaphoreType.DMA,
        ],
        compiler_params=pltpu.CompilerParams(use_tc_tiling_on_sc=False),
    )
    def k(table_hbm, idx_hbm, out_hbm, idxbuf, rows2, rows_x, gsem, wsem):
        wid = lax.axis_index("s") * _NC + lax.axis_index("c")
        row0 = wid * _CPW
        pltpu.sync_copy(idx_hbm.at[pl.ds(row0, _CPW)],
                        idxbuf.at[pl.ds(0, _CPW)])

        def super_body(j, carry):
            p = lax.rem(j, 2)

            @pl.when(j >= 2)
            def _():
                pltpu.make_async_copy(
                    out_hbm.at[pl.ds(0, _SROWS)], rows2.at[p], wsem).wait()

            handles = []
            for b in range(_SUP):
                handles.append(pltpu.async_copy(
                    table_hbm.at[idxbuf.at[j * _SUP + b]],
                    rows2.at[p, pl.ds(b * _CH, _CH)], gsem))
            for h in handles:
                h.wait()
            pltpu.async_copy(
                rows2.at[p],
                out_hbm.at[pl.ds((row0 + j * _SUP) * _CH, _SROWS)], wsem)
            return carry

        lax.fori_loop(0, _NSUP, super_body, 0)
        pltpu.make_async_copy(
            out_hbm.at[pl.ds(0, _SROWS)], rows2.at[0], wsem).wait()
        pltpu.make_async_copy(
            out_hbm.at[pl.ds(0, _SROWS)], rows2.at[1], wsem).wait()

        @pl.when(wid < _XTRA)
        def _():
            xrow = _NW * _CPW + wid
            pltpu.sync_copy(idx_hbm.at[xrow], idxbuf.at[_CPW])
            pltpu.async_copy(
                table_hbm.at[idxbuf.at[_CPW]], rows_x, gsem).wait()
            pltpu.sync_copy(rows_x, out_hbm.at[pl.ds(xrow * _CH, _CH)])

    return k(table, idx2d)


def _sc_scatter(msg, dst2d):
    mesh = plsc.VectorSubcoreMesh(core_axis_name="c", subcore_axis_name="s")

    @functools.partial(
        pl.kernel,
        mesh=mesh,
        out_type=jax.ShapeDtypeStruct((_NC, _N, _D), jnp.float32),
        scratch_types=[
            pltpu.VMEM((_CPW + 1, _CH), jnp.int32),
            pltpu.VMEM((2, _SROWS, _D), jnp.float32),
            pltpu.VMEM((_CH, _D), jnp.float32),
            pltpu.VMEM((_ZR, _D), jnp.float32),
            pltpu.VMEM_SHARED((_N, _D), jnp.float32),
            pltpu.SemaphoreType.DMA,
        ],
        compiler_params=pltpu.CompilerParams(use_tc_tiling_on_sc=False),
    )
    def k(msg_hbm, dst_hbm, out_hbm, idxbuf, msg2, msg_x, z_v, acc, lsem):
        cid = lax.axis_index("c")
        sid = lax.axis_index("s")
        wid = sid * _NC + cid
        row0 = wid * _CPW
        zero = jnp.zeros((_D,), jnp.float32)

        def zb(i, carry):
            z_v[i, :] = zero
            return carry

        lax.fori_loop(0, _ZR, zb, 0)
        pltpu.sync_copy(z_v, acc.at[pl.ds(sid * _ZR, _ZR)])
        pltpu.sync_copy(dst_hbm.at[pl.ds(row0, _CPW)],
                        idxbuf.at[pl.ds(0, _CPW)])
        plsc.subcore_barrier()

        pltpu.async_copy(msg_hbm.at[pl.ds(row0 * _CH, _SROWS)],
                         msg2.at[0], lsem)

        def super_body(j, carry):
            p = lax.rem(j, 2)
            pltpu.make_async_copy(
                msg_hbm.at[pl.ds(0, _SROWS)], msg2.at[p], lsem).wait()

            @pl.when(j < _NSUP - 1)
            def _():
                pltpu.async_copy(
                    msg_hbm.at[pl.ds((row0 + (j + 1) * _SUP) * _CH, _SROWS)],
                    msg2.at[1 - p], lsem)

            for b in range(_SUP):
                pltpu.sync_copy(msg2.at[p, pl.ds(b * _CH, _CH)],
                                acc.at[idxbuf.at[j * _SUP + b]], add=True)
            return carry

        lax.fori_loop(0, _NSUP, super_body, 0)

        @pl.when(wid < _XTRA)
        def _():
            xrow = _NW * _CPW + wid
            pltpu.sync_copy(dst_hbm.at[xrow], idxbuf.at[_CPW])
            pltpu.sync_copy(msg_hbm.at[pl.ds(xrow * _CH, _CH)], msg_x)
            pltpu.sync_copy(msg_x, acc.at[idxbuf.at[_CPW]], add=True)

        plsc.subcore_barrier()
        pltpu.sync_copy(acc.at[pl.ds(sid * _ZR, _ZR)],
                        out_hbm.at[cid, pl.ds(sid * _ZR, _ZR)])

    return k(msg, dst2d)


def _tc_msg(eap, xjp, W1p, b1p, W2p, b2p, W3p, b3p, Rp, Sp):
    _TR = _TE // 8

    def body(eap_ref, xjp_ref, w1p, b1r, w2p, b2r, w3p, b3r, rp, sp,
             out_ref):
        h = jnp.dot(eap_ref[...], w1p[...], preferred_element_type=jnp.float32)
        h = _gelu_exact(h + b1r[...])
        h = jnp.dot(h, w2p[...], preferred_element_type=jnp.float32)
        h = _gelu_exact(h + b2r[...])
        w3v = w3p[...]
        wp = jnp.concatenate(
            [jnp.dot(h[:, 64 * k:64 * (k + 1)], w3v,
                     preferred_element_type=jnp.float32)
             for k in range(8)], axis=1)
        wp = wp + b3r[...]
        xrep = jnp.dot(xjp_ref[...], rp[...],
                       preferred_element_type=jnp.float32)
        out_ref[...] = jnp.dot(xrep * wp, sp[...],
                               preferred_element_type=jnp.float32)

    return pl.pallas_call(
        body,
        grid=(_E // _TE,),
        in_specs=[
            pl.BlockSpec((_TR, 128), lambda i: (i, 0)),
            pl.BlockSpec((_TR, 128), lambda i: (i, 0)),
            pl.BlockSpec((128, 8 * _KD), lambda i: (0, 0)),
            pl.BlockSpec((1, 8 * _KD), lambda i: (0, 0)),
            pl.BlockSpec((8 * _KD, 8 * _KD), lambda i: (0, 0)),
            pl.BlockSpec((1, 8 * _KD), lambda i: (0, 0)),
            pl.BlockSpec((_KD, _D * _D), lambda i: (0, 0)),
            pl.BlockSpec((1, 8 * _D * _D), lambda i: (0, 0)),
            pl.BlockSpec((128, 8 * _D * _D), lambda i: (0, 0)),
            pl.BlockSpec((8 * _D * _D, 128), lambda i: (0, 0)),
        ],
        out_specs=pl.BlockSpec((_TR, 128), lambda i: (i, 0)),
        out_shape=jax.ShapeDtypeStruct((_E * _D // 128, 128), jnp.float32),
    )(eap, xjp, W1p, b1p, W2p, b2p, W3p, b3p, Rp, Sp)


def _tc_final(x, parts, root, bias, use_gelu):
    def body(x_ref, p_ref, r_ref, b_ref, out_ref):
        out = p_ref[0] + p_ref[1] + b_ref[...]
        out = out + jnp.dot(x_ref[...], r_ref[...],
                            preferred_element_type=jnp.float32)
        if use_gelu:
            out = _gelu_exact(out)
        out_ref[...] = out

    return pl.pallas_call(
        body,
        out_shape=jax.ShapeDtypeStruct((_N, _D), jnp.float32),
    )(x, parts, root, bias)


def kernel(nodes, edge_index, edge_attr, W1, b1, W2, b2, W3, b3,
           root1, bias1, root2, bias2):
    src2d = edge_index[0].astype(jnp.int32).reshape(_NCK, _CH)
    dst2d = edge_index[1].astype(jnp.int32).reshape(_NCK, _CH)
    eye8 = jnp.eye(8, dtype=jnp.float32)
    W1p = jnp.kron(eye8, W1)
    W2p = jnp.kron(eye8, W2)
    W3p = W3
    b1p = jnp.tile(b1, 8).reshape(1, 8 * _KD)
    b2p = jnp.tile(b2, 8).reshape(1, 8 * _KD)
    b3p = jnp.tile(b3, 8).reshape(1, 8 * _D * _D)
    la = jnp.arange(128, dtype=jnp.int32)[:, None]
    lb = jnp.arange(8 * _D * _D, dtype=jnp.int32)[None, :]
    Rp = (la == _D * (lb // (_D * _D))
          + (lb % (_D * _D)) // _D).astype(jnp.float32)
    lc = jnp.arange(8 * _D * _D, dtype=jnp.int32)[:, None]
    ld = jnp.arange(128, dtype=jnp.int32)[None, :]
    Sp = ((lc // (_D * _D) == ld // _D)
          & (lc % _D == ld % _D)).astype(jnp.float32)
    eap = edge_attr.reshape(_E * _ED // 128, 128)
    bias1r = bias1.reshape(1, _D)
    bias2r = bias2.reshape(1, _D)

    def gno_pass(x, root, biasr, use_gelu):
        xj = _sc_gather(x, src2d)
        xjp = xj.reshape(_E * _D // 128, 128)
        msgp = _tc_msg(eap, xjp, W1p, b1p, W2p, b2p, W3p, b3p, Rp, Sp)
        parts = _sc_scatter(msgp.reshape(_E, _D), dst2d)
        return _tc_final(x, parts, root, biasr, use_gelu)

    h = gno_pass(nodes, root1, bias1r, True)
    return gno_pass(h, root2, bias2r, False)


# speedup vs baseline: 6.5478x; 1.0121x over previous
import functools

import jax
import jax.numpy as jnp
from jax import lax
from jax.experimental import pallas as pl
from jax.experimental.pallas import tpu as pltpu
from jax.experimental.pallas import tpu_sc as plsc

_N = 10000
_E = 320000
_D = 16
_ED = 16
_KD = 64

_NC = 2
_NS = 16
_NW = _NC * _NS
_CH = 128
_ZR = _N // _NS

_EH = _E // 2
_NCKH = _EH // _CH
_TE = 3200
_TR = _TE // 8

_INV_SQRT2 = 0.7071067811865476


def _gelu_exact(x):
    return 0.5 * x * (1.0 + lax.erf(x * _INV_SQRT2))


def _sc_gather(table, idx2d, ncks):
    cpw = ncks // _NW
    xtra = ncks - cpw * _NW
    sup = 6 if cpw % 6 == 0 else 3
    nsup = cpw // sup
    srows = sup * _CH
    mesh = plsc.VectorSubcoreMesh(core_axis_name="c", subcore_axis_name="s")

    @functools.partial(
        pl.kernel,
        mesh=mesh,
        out_type=jax.ShapeDtypeStruct((ncks * _CH, _D), jnp.float32),
        scratch_types=[
            pltpu.VMEM((cpw + 1, _CH), jnp.int32),
            pltpu.VMEM((2, srows, _D), jnp.float32),
            pltpu.VMEM((_CH, _D), jnp.float32),
            pltpu.SemaphoreType.DMA,
            pltpu.SemaphoreType.DMA,
        ],
        compiler_params=pltpu.CompilerParams(use_tc_tiling_on_sc=False),
    )
    def k(table_hbm, idx_hbm, out_hbm, idxbuf, rows2, rows_x, gsem, wsem):
        wid = lax.axis_index("s") * _NC + lax.axis_index("c")
        row0 = wid * cpw
        pltpu.sync_copy(idx_hbm.at[pl.ds(row0, cpw)],
                        idxbuf.at[pl.ds(0, cpw)])

        def super_body(j, carry):
            p = lax.rem(j, 2)

            @pl.when(j >= 2)
            def _():
                pltpu.make_async_copy(
                    out_hbm.at[pl.ds(0, srows)], rows2.at[p], wsem).wait()

            handles = []
            for b in range(sup):
                handles.append(pltpu.async_copy(
                    table_hbm.at[idxbuf.at[j * sup + b]],
                    rows2.at[p, pl.ds(b * _CH, _CH)], gsem))
            for h in handles:
                h.wait()
            pltpu.async_copy(
                rows2.at[p],
                out_hbm.at[pl.ds((row0 + j * sup) * _CH, srows)], wsem)
            return carry

        lax.fori_loop(0, nsup, super_body, 0)
        pltpu.make_async_copy(
            out_hbm.at[pl.ds(0, srows)], rows2.at[0], wsem).wait()
        pltpu.make_async_copy(
            out_hbm.at[pl.ds(0, srows)], rows2.at[1], wsem).wait()

        @pl.when(wid < xtra)
        def _():
            xrow = _NW * cpw + wid
            pltpu.sync_copy(idx_hbm.at[xrow], idxbuf.at[cpw])
            pltpu.async_copy(
                table_hbm.at[idxbuf.at[cpw]], rows_x, gsem).wait()
            pltpu.sync_copy(rows_x, out_hbm.at[pl.ds(xrow * _CH, _CH)])

    return k(table, idx2d)


def _sc_scatter(msg, dst2d, ncks):
    cpw = ncks // _NW
    xtra = ncks - cpw * _NW
    sup = 6 if cpw % 6 == 0 else 3
    nsup = cpw // sup
    srows = sup * _CH
    mesh = plsc.VectorSubcoreMesh(core_axis_name="c", subcore_axis_name="s")

    @functools.partial(
        pl.kernel,
        mesh=mesh,
        out_type=jax.ShapeDtypeStruct((_NC, _N, _D), jnp.float32),
        scratch_types=[
            pltpu.VMEM((cpw + 1, _CH), jnp.int32),
            pltpu.VMEM((2, srows, _D), jnp.float32),
            pltpu.VMEM((_CH, _D), jnp.float32),
            pltpu.VMEM((_ZR, _D), jnp.float32),
            pltpu.VMEM_SHARED((_N, _D), jnp.float32),
            pltpu.SemaphoreType.DMA,
        ],
        compiler_params=pltpu.CompilerParams(use_tc_tiling_on_sc=False),
    )
    def k(msg_hbm, dst_hbm, out_hbm, idxbuf, msg2, msg_x, z_v, acc, lsem):
        cid = lax.axis_index("c")
        sid = lax.axis_index("s")
        wid = sid * _NC + cid
        row0 = wid * cpw
        zero = jnp.zeros((_D,), jnp.float32)

        def zb(i, carry):
            z_v[i, :] = zero
            return carry

        lax.fori_loop(0, _ZR, zb, 0)
        pltpu.sync_copy(z_v, acc.at[pl.ds(sid * _ZR, _ZR)])
        pltpu.sync_copy(dst_hbm.at[pl.ds(row0, cpw)],
                        idxbuf.at[pl.ds(0, cpw)])
        plsc.subcore_barrier()

        pltpu.async_copy(msg_hbm.at[pl.ds(row0 * _CH, srows)],
                         msg2.at[0], lsem)

        def super_body(j, carry):
            p = lax.rem(j, 2)
            pltpu.make_async_copy(
                msg_hbm.at[pl.ds(0, srows)], msg2.at[p], lsem).wait()

            @pl.when(j < nsup - 1)
            def _():
                pltpu.async_copy(
                    msg_hbm.at[pl.ds((row0 + (j + 1) * sup) * _CH, srows)],
                    msg2.at[1 - p], lsem)

            for b in range(sup):
                pltpu.sync_copy(msg2.at[p, pl.ds(b * _CH, _CH)],
                                acc.at[idxbuf.at[j * sup + b]], add=True)
            return carry

        lax.fori_loop(0, nsup, super_body, 0)

        @pl.when(wid < xtra)
        def _():
            xrow = _NW * cpw + wid
            pltpu.sync_copy(dst_hbm.at[xrow], idxbuf.at[cpw])
            pltpu.sync_copy(msg_hbm.at[pl.ds(xrow * _CH, _CH)], msg_x)
            pltpu.sync_copy(msg_x, acc.at[idxbuf.at[cpw]], add=True)

        plsc.subcore_barrier()
        pltpu.sync_copy(acc.at[pl.ds(sid * _ZR, _ZR)],
                        out_hbm.at[cid, pl.ds(sid * _ZR, _ZR)])

    return k(msg, dst2d)


def _tc_msg(eap, xjp, W1p, b1p, W2p, b2p, W3, b3p, Rp, Sp, blk_off, n_edges):
    def body(eap_ref, xjp_ref, w1p, b1r, w2p, b2r, w3, b3r, rp, sp,
             out_ref):
        h = jnp.dot(eap_ref[...], w1p[...], preferred_element_type=jnp.float32)
        h = _gelu_exact(h + b1r[...])
        h = jnp.dot(h, w2p[...], preferred_element_type=jnp.float32)
        h = _gelu_exact(h + b2r[...])
        w3v = w3[...]
        wp = jnp.concatenate(
            [jnp.dot(h[:, 64 * k:64 * (k + 1)], w3v,
                     preferred_element_type=jnp.float32)
             for k in range(8)], axis=1)
        wp = wp + b3r[...]
        xrep = jnp.dot(xjp_ref[...], rp[...],
                       preferred_element_type=jnp.float32)
        out_ref[...] = jnp.dot(xrep * wp, sp[...],
                               preferred_element_type=jnp.float32)

    return pl.pallas_call(
        body,
        grid=(n_edges // _TE,),
        in_specs=[
            pl.BlockSpec((_TR, 128), lambda i: (i + blk_off, 0)),
            pl.BlockSpec((_TR, 128), lambda i: (i, 0)),
            pl.BlockSpec((128, 8 * _KD), lambda i: (0, 0)),
            pl.BlockSpec((1, 8 * _KD), lambda i: (0, 0)),
            pl.BlockSpec((8 * _KD, 8 * _KD), lambda i: (0, 0)),
            pl.BlockSpec((1, 8 * _KD), lambda i: (0, 0)),
            pl.BlockSpec((_KD, _D * _D), lambda i: (0, 0)),
            pl.BlockSpec((1, 8 * _D * _D), lambda i: (0, 0)),
            pl.BlockSpec((128, 8 * _D * _D), lambda i: (0, 0)),
            pl.BlockSpec((8 * _D * _D, 128), lambda i: (0, 0)),
        ],
        out_specs=pl.BlockSpec((_TR, 128), lambda i: (i, 0)),
        out_shape=jax.ShapeDtypeStruct((n_edges * _D // 128, 128),
                                       jnp.float32),
    )(eap, xjp, W1p, b1p, W2p, b2p, W3, b3p, Rp, Sp)


def _tc_final(x, parts_a, parts_b, root, bias, use_gelu):
    def body(x_ref, pa_ref, pb_ref, r_ref, b_ref, out_ref):
        out = pa_ref[0] + pa_ref[1] + pb_ref[0] + pb_ref[1] + b_ref[...]
        out = out + jnp.dot(x_ref[...], r_ref[...],
                            preferred_element_type=jnp.float32)
        if use_gelu:
            out = _gelu_exact(out)
        out_ref[...] = out

    return pl.pallas_call(
        body,
        out_shape=jax.ShapeDtypeStruct((_N, _D), jnp.float32),
    )(x, parts_a, parts_b, root, bias)


def kernel(nodes, edge_index, edge_attr, W1, b1, W2, b2, W3, b3,
           root1, bias1, root2, bias2):
    src2d = edge_index[0].astype(jnp.int32).reshape(_E // _CH, _CH)
    dst2d = edge_index[1].astype(jnp.int32).reshape(_E // _CH, _CH)
    src_h = (src2d[:_NCKH], src2d[_NCKH:])
    dst_h = (dst2d[:_NCKH], dst2d[_NCKH:])
    eye8 = jnp.eye(8, dtype=jnp.float32)
    W1p = jnp.kron(eye8, W1)
    W2p = jnp.kron(eye8, W2)
    b1p = jnp.tile(b1, 8).reshape(1, 8 * _KD)
    b2p = jnp.tile(b2, 8).reshape(1, 8 * _KD)
    b3p = jnp.tile(b3, 8).reshape(1, 8 * _D * _D)
    la = jnp.arange(128, dtype=jnp.int32)[:, None]
    lb = jnp.arange(8 * _D * _D, dtype=jnp.int32)[None, :]
    Rp = (la == _D * (lb // (_D * _D))
          + (lb % (_D * _D)) // _D).astype(jnp.float32)
    lc = jnp.arange(8 * _D * _D, dtype=jnp.int32)[:, None]
    ld = jnp.arange(128, dtype=jnp.int32)[None, :]
    Sp = ((lc // (_D * _D) == ld // _D)
          & (lc % _D == ld % _D)).astype(jnp.float32)
    eap = edge_attr.reshape(_E * _ED // 128, 128)
    bias1r = bias1.reshape(1, _D)
    bias2r = bias2.reshape(1, _D)
    hblk = _EH * _D // 128 // _TR

    def gno_pass(x, root, biasr, use_gelu):
        xj = [_sc_gather(x, src_h[i], _NCKH) for i in range(2)]
        msg = [_tc_msg(eap, xj[i].reshape(_EH * _D // 128, 128),
                       W1p, b1p, W2p, b2p, W3, b3p, Rp, Sp,
                       i * hblk, _EH) for i in range(2)]
        parts = [_sc_scatter(msg[i].reshape(_EH, _D), dst_h[i], _NCKH)
                 for i in range(2)]
        return _tc_final(x, parts[0], parts[1], root, biasr, use_gelu)

    h = gno_pass(nodes, root1, bias1r, True)
    return gno_pass(h, root2, bias2r, False)


# speedup vs baseline: 6.7197x; 1.0263x over previous
import functools

import jax
import jax.numpy as jnp
from jax import lax
from jax.experimental import pallas as pl
from jax.experimental.pallas import tpu as pltpu
from jax.experimental.pallas import tpu_sc as plsc

_N = 10000
_E = 320000
_D = 16
_ED = 16
_KD = 64

_NC = 2
_NS = 16
_NW = _NC * _NS
_CH = 128
_ZR = _N // _NS

_EH = _E // 2
_NCKH = _EH // _CH
_TE = 3200
_TR = _TE // 8

_INV_SQRT2 = 0.7071067811865476


def _gelu_exact(x):
    return 0.5 * x * (1.0 + lax.erf(x * _INV_SQRT2))


def _sc_gather(table, idx2d, ncks):
    cpw = ncks // _NW
    xtra = ncks - cpw * _NW
    sup = 6 if cpw % 6 == 0 else 3
    nsup = cpw // sup
    srows = sup * _CH
    mesh = plsc.VectorSubcoreMesh(core_axis_name="c", subcore_axis_name="s")

    @functools.partial(
        pl.kernel,
        mesh=mesh,
        out_type=jax.ShapeDtypeStruct((ncks * _CH, _D), jnp.float32),
        scratch_types=[
            pltpu.VMEM((cpw + 1, _CH), jnp.int32),
            pltpu.VMEM((2, srows, _D), jnp.float32),
            pltpu.VMEM((_CH, _D), jnp.float32),
            pltpu.SemaphoreType.DMA,
            pltpu.SemaphoreType.DMA,
        ],
        compiler_params=pltpu.CompilerParams(use_tc_tiling_on_sc=False),
    )
    def k(table_hbm, idx_hbm, out_hbm, idxbuf, rows2, rows_x, gsem, wsem):
        wid = lax.axis_index("s") * _NC + lax.axis_index("c")
        row0 = wid * cpw
        pltpu.sync_copy(idx_hbm.at[pl.ds(row0, cpw)],
                        idxbuf.at[pl.ds(0, cpw)])

        def super_body(j, carry):
            p = lax.rem(j, 2)

            @pl.when(j >= 2)
            def _():
                pltpu.make_async_copy(
                    out_hbm.at[pl.ds(0, srows)], rows2.at[p], wsem).wait()

            handles = []
            for b in range(sup):
                handles.append(pltpu.async_copy(
                    table_hbm.at[idxbuf.at[j * sup + b]],
                    rows2.at[p, pl.ds(b * _CH, _CH)], gsem))
            for h in handles:
                h.wait()
            pltpu.async_copy(
                rows2.at[p],
                out_hbm.at[pl.ds((row0 + j * sup) * _CH, srows)], wsem)
            return carry

        lax.fori_loop(0, nsup, super_body, 0)
        pltpu.make_async_copy(
            out_hbm.at[pl.ds(0, srows)], rows2.at[0], wsem).wait()
        pltpu.make_async_copy(
            out_hbm.at[pl.ds(0, srows)], rows2.at[1], wsem).wait()

        @pl.when(wid < xtra)
        def _():
            xrow = _NW * cpw + wid
            pltpu.sync_copy(idx_hbm.at[xrow], idxbuf.at[cpw])
            pltpu.async_copy(
                table_hbm.at[idxbuf.at[cpw]], rows_x, gsem).wait()
            pltpu.sync_copy(rows_x, out_hbm.at[pl.ds(xrow * _CH, _CH)])

    return k(table, idx2d)


def _sc_scatter(msg, dst2d, ncks):
    cpw = ncks // _NW
    xtra = ncks - cpw * _NW
    sup = 6 if cpw % 6 == 0 else 3
    nsup = cpw // sup
    srows = sup * _CH
    mesh = plsc.VectorSubcoreMesh(core_axis_name="c", subcore_axis_name="s")

    @functools.partial(
        pl.kernel,
        mesh=mesh,
        out_type=jax.ShapeDtypeStruct((_NC, _N, _D), jnp.float32),
        scratch_types=[
            pltpu.VMEM((cpw + 1, _CH), jnp.int32),
            pltpu.VMEM((2, srows, _D), jnp.float32),
            pltpu.VMEM((_CH, _D), jnp.float32),
            pltpu.VMEM((_ZR, _D), jnp.float32),
            pltpu.VMEM_SHARED((_N, _D), jnp.float32),
            pltpu.SemaphoreType.DMA,
        ],
        compiler_params=pltpu.CompilerParams(use_tc_tiling_on_sc=False),
    )
    def k(msg_hbm, dst_hbm, out_hbm, idxbuf, msg2, msg_x, z_v, acc, lsem):
        cid = lax.axis_index("c")
        sid = lax.axis_index("s")
        wid = sid * _NC + cid
        row0 = wid * cpw
        zero = jnp.zeros((_D,), jnp.float32)

        def zb(i, carry):
            z_v[i, :] = zero
            return carry

        lax.fori_loop(0, _ZR, zb, 0)
        pltpu.sync_copy(z_v, acc.at[pl.ds(sid * _ZR, _ZR)])
        pltpu.sync_copy(dst_hbm.at[pl.ds(row0, cpw)],
                        idxbuf.at[pl.ds(0, cpw)])
        plsc.subcore_barrier()

        pltpu.async_copy(msg_hbm.at[pl.ds(row0 * _CH, srows)],
                         msg2.at[0], lsem)

        def super_body(j, carry):
            p = lax.rem(j, 2)
            pltpu.make_async_copy(
                msg_hbm.at[pl.ds(0, srows)], msg2.at[p], lsem).wait()

            @pl.when(j < nsup - 1)
            def _():
                pltpu.async_copy(
                    msg_hbm.at[pl.ds((row0 + (j + 1) * sup) * _CH, srows)],
                    msg2.at[1 - p], lsem)

            for b in range(sup):
                pltpu.sync_copy(msg2.at[p, pl.ds(b * _CH, _CH)],
                                acc.at[idxbuf.at[j * sup + b]], add=True)
            return carry

        lax.fori_loop(0, nsup, super_body, 0)

        @pl.when(wid < xtra)
        def _():
            xrow = _NW * cpw + wid
            pltpu.sync_copy(dst_hbm.at[xrow], idxbuf.at[cpw])
            pltpu.sync_copy(msg_hbm.at[pl.ds(xrow * _CH, _CH)], msg_x)
            pltpu.sync_copy(msg_x, acc.at[idxbuf.at[cpw]], add=True)

        plsc.subcore_barrier()
        pltpu.sync_copy(acc.at[pl.ds(sid * _ZR, _ZR)],
                        out_hbm.at[cid, pl.ds(sid * _ZR, _ZR)])

    return k(msg, dst2d)


def _tc_msg(eap, xjp, W1p, b1p, W2p, b2p, W3, b3p, Rp, Sp, blk_off, n_edges):
    def body(eap_ref, xjp_ref, w1p, b1r, w2p, b2r, w3, b3r, rp, sp,
             out_ref):
        h = jnp.dot(eap_ref[...], w1p[...], preferred_element_type=jnp.float32)
        h = _gelu_exact(h + b1r[...])
        h = jnp.dot(h, w2p[...], preferred_element_type=jnp.float32)
        h = _gelu_exact(h + b2r[...])
        w3v = w3[...]
        b3v = b3r[...]
        xjv = xjp_ref[...]
        rpv = rp[...]
        spv = sp[...]
        acc = jnp.zeros((h.shape[0], 128), jnp.float32)
        for k in range(8):
            wp_k = jnp.dot(h[:, 64 * k:64 * (k + 1)], w3v,
                           preferred_element_type=jnp.float32)
            wp_k = wp_k + b3v[:, 256 * k:256 * (k + 1)]
            xrep_k = jnp.dot(xjv, rpv[:, 256 * k:256 * (k + 1)],
                             preferred_element_type=jnp.float32)
            acc = acc + jnp.dot(xrep_k * wp_k,
                                spv[256 * k:256 * (k + 1), :],
                                preferred_element_type=jnp.float32)
        out_ref[...] = acc

    return pl.pallas_call(
        body,
        grid=(n_edges // _TE,),
        in_specs=[
            pl.BlockSpec((_TR, 128), lambda i: (i + blk_off, 0)),
            pl.BlockSpec((_TR, 128), lambda i: (i, 0)),
            pl.BlockSpec((128, 8 * _KD), lambda i: (0, 0)),
            pl.BlockSpec((1, 8 * _KD), lambda i: (0, 0)),
            pl.BlockSpec((8 * _KD, 8 * _KD), lambda i: (0, 0)),
            pl.BlockSpec((1, 8 * _KD), lambda i: (0, 0)),
            pl.BlockSpec((_KD, _D * _D), lambda i: (0, 0)),
            pl.BlockSpec((1, 8 * _D * _D), lambda i: (0, 0)),
            pl.BlockSpec((128, 8 * _D * _D), lambda i: (0, 0)),
            pl.BlockSpec((8 * _D * _D, 128), lambda i: (0, 0)),
        ],
        out_specs=pl.BlockSpec((_TR, 128), lambda i: (i, 0)),
        out_shape=jax.ShapeDtypeStruct((n_edges * _D // 128, 128),
                                       jnp.float32),
    )(eap, xjp, W1p, b1p, W2p, b2p, W3, b3p, Rp, Sp)


def _tc_final(x, parts_a, parts_b, root, bias, use_gelu):
    def body(x_ref, pa_ref, pb_ref, r_ref, b_ref, out_ref):
        out = pa_ref[0] + pa_ref[1] + pb_ref[0] + pb_ref[1] + b_ref[...]
        out = out + jnp.dot(x_ref[...], r_ref[...],
                            preferred_element_type=jnp.float32)
        if use_gelu:
            out = _gelu_exact(out)
        out_ref[...] = out

    return pl.pallas_call(
        body,
        out_shape=jax.ShapeDtypeStruct((_N, _D), jnp.float32),
    )(x, parts_a, parts_b, root, bias)


def kernel(nodes, edge_index, edge_attr, W1, b1, W2, b2, W3, b3,
           root1, bias1, root2, bias2):
    src2d = edge_index[0].astype(jnp.int32).reshape(_E // _CH, _CH)
    dst2d = edge_index[1].astype(jnp.int32).reshape(_E // _CH, _CH)
    src_h = (src2d[:_NCKH], src2d[_NCKH:])
    dst_h = (dst2d[:_NCKH], dst2d[_NCKH:])
    eye8 = jnp.eye(8, dtype=jnp.float32)
    W1p = jnp.kron(eye8, W1)
    W2p = jnp.kron(eye8, W2)
    b1p = jnp.tile(b1, 8).reshape(1, 8 * _KD)
    b2p = jnp.tile(b2, 8).reshape(1, 8 * _KD)
    b3p = jnp.tile(b3, 8).reshape(1, 8 * _D * _D)
    la = jnp.arange(128, dtype=jnp.int32)[:, None]
    lb = jnp.arange(8 * _D * _D, dtype=jnp.int32)[None, :]
    Rp = (la == _D * (lb // (_D * _D))
          + (lb % (_D * _D)) // _D).astype(jnp.float32)
    lc = jnp.arange(8 * _D * _D, dtype=jnp.int32)[:, None]
    ld = jnp.arange(128, dtype=jnp.int32)[None, :]
    Sp = ((lc // (_D * _D) == ld // _D)
          & (lc % _D == ld % _D)).astype(jnp.float32)
    eap = edge_attr.reshape(_E * _ED // 128, 128)
    bias1r = bias1.reshape(1, _D)
    bias2r = bias2.reshape(1, _D)
    hblk = _EH * _D // 128 // _TR

    def gno_pass(x, root, biasr, use_gelu):
        xj = [_sc_gather(x, src_h[i], _NCKH) for i in range(2)]
        msg = [_tc_msg(eap, xj[i].reshape(_EH * _D // 128, 128),
                       W1p, b1p, W2p, b2p, W3, b3p, Rp, Sp,
                       i * hblk, _EH) for i in range(2)]
        parts = [_sc_scatter(msg[i].reshape(_EH, _D), dst_h[i], _NCKH)
                 for i in range(2)]
        return _tc_final(x, parts[0], parts[1], root, biasr, use_gelu)

    h = gno_pass(nodes, root1, bias1r, True)
    return gno_pass(h, root2, bias2r, False)


# speedup vs baseline: 7.0794x; 1.0535x over previous
import functools

import jax
import jax.numpy as jnp
from jax import lax
from jax.experimental import pallas as pl
from jax.experimental.pallas import tpu as pltpu
from jax.experimental.pallas import tpu_sc as plsc

_N = 10000
_E = 320000
_D = 16
_ED = 16
_KD = 64

_NC = 2
_NS = 16
_NW = _NC * _NS
_CH = 128
_ZR = _N // _NS

_EH = _E // 2
_NCKH = _EH // _CH
_TE = 6400
_TR = _TE // 8

_INV_SQRT2 = 0.7071067811865476


def _gelu_exact(x):
    return 0.5 * x * (1.0 + lax.erf(x * _INV_SQRT2))


def _sc_gather(table, idx2d, ncks):
    cpw = ncks // _NW
    xtra = ncks - cpw * _NW
    sup = 6 if cpw % 6 == 0 else 3
    nsup = cpw // sup
    srows = sup * _CH
    mesh = plsc.VectorSubcoreMesh(core_axis_name="c", subcore_axis_name="s")

    @functools.partial(
        pl.kernel,
        mesh=mesh,
        out_type=jax.ShapeDtypeStruct((ncks * _CH, _D), jnp.float32),
        scratch_types=[
            pltpu.VMEM((cpw + 1, _CH), jnp.int32),
            pltpu.VMEM((2, srows, _D), jnp.float32),
            pltpu.VMEM((_CH, _D), jnp.float32),
            pltpu.SemaphoreType.DMA,
            pltpu.SemaphoreType.DMA,
        ],
        compiler_params=pltpu.CompilerParams(use_tc_tiling_on_sc=False),
    )
    def k(table_hbm, idx_hbm, out_hbm, idxbuf, rows2, rows_x, gsem, wsem):
        wid = lax.axis_index("s") * _NC + lax.axis_index("c")
        row0 = wid * cpw
        pltpu.sync_copy(idx_hbm.at[pl.ds(row0, cpw)],
                        idxbuf.at[pl.ds(0, cpw)])

        def super_body(j, carry):
            p = lax.rem(j, 2)

            @pl.when(j >= 2)
            def _():
                pltpu.make_async_copy(
                    out_hbm.at[pl.ds(0, srows)], rows2.at[p], wsem).wait()

            handles = []
            for b in range(sup):
                handles.append(pltpu.async_copy(
                    table_hbm.at[idxbuf.at[j * sup + b]],
                    rows2.at[p, pl.ds(b * _CH, _CH)], gsem))
            for h in handles:
                h.wait()
            pltpu.async_copy(
                rows2.at[p],
                out_hbm.at[pl.ds((row0 + j * sup) * _CH, srows)], wsem)
            return carry

        lax.fori_loop(0, nsup, super_body, 0)
        pltpu.make_async_copy(
            out_hbm.at[pl.ds(0, srows)], rows2.at[0], wsem).wait()
        pltpu.make_async_copy(
            out_hbm.at[pl.ds(0, srows)], rows2.at[1], wsem).wait()

        @pl.when(wid < xtra)
        def _():
            xrow = _NW * cpw + wid
            pltpu.sync_copy(idx_hbm.at[xrow], idxbuf.at[cpw])
            pltpu.async_copy(
                table_hbm.at[idxbuf.at[cpw]], rows_x, gsem).wait()
            pltpu.sync_copy(rows_x, out_hbm.at[pl.ds(xrow * _CH, _CH)])

    return k(table, idx2d)


def _sc_scatter(msg, dst2d, ncks):
    cpw = ncks // _NW
    xtra = ncks - cpw * _NW
    sup = 6 if cpw % 6 == 0 else 3
    nsup = cpw // sup
    srows = sup * _CH
    mesh = plsc.VectorSubcoreMesh(core_axis_name="c", subcore_axis_name="s")

    @functools.partial(
        pl.kernel,
        mesh=mesh,
        out_type=jax.ShapeDtypeStruct((_NC, _N, _D), jnp.float32),
        scratch_types=[
            pltpu.VMEM((cpw + 1, _CH), jnp.int32),
            pltpu.VMEM((2, srows, _D), jnp.float32),
            pltpu.VMEM((_CH, _D), jnp.float32),
            pltpu.VMEM((_ZR, _D), jnp.float32),
            pltpu.VMEM_SHARED((_N, _D), jnp.float32),
            pltpu.SemaphoreType.DMA,
        ],
        compiler_params=pltpu.CompilerParams(use_tc_tiling_on_sc=False),
    )
    def k(msg_hbm, dst_hbm, out_hbm, idxbuf, msg2, msg_x, z_v, acc, lsem):
        cid = lax.axis_index("c")
        sid = lax.axis_index("s")
        wid = sid * _NC + cid
        row0 = wid * cpw
        zero = jnp.zeros((_D,), jnp.float32)

        def zb(i, carry):
            z_v[i, :] = zero
            return carry

        lax.fori_loop(0, _ZR, zb, 0)
        pltpu.sync_copy(z_v, acc.at[pl.ds(sid * _ZR, _ZR)])
        pltpu.sync_copy(dst_hbm.at[pl.ds(row0, cpw)],
                        idxbuf.at[pl.ds(0, cpw)])
        plsc.subcore_barrier()

        pltpu.async_copy(msg_hbm.at[pl.ds(row0 * _CH, srows)],
                         msg2.at[0], lsem)

        def super_body(j, carry):
            p = lax.rem(j, 2)
            pltpu.make_async_copy(
                msg_hbm.at[pl.ds(0, srows)], msg2.at[p], lsem).wait()

            @pl.when(j < nsup - 1)
            def _():
                pltpu.async_copy(
                    msg_hbm.at[pl.ds((row0 + (j + 1) * sup) * _CH, srows)],
                    msg2.at[1 - p], lsem)

            for b in range(sup):
                pltpu.sync_copy(msg2.at[p, pl.ds(b * _CH, _CH)],
                                acc.at[idxbuf.at[j * sup + b]], add=True)
            return carry

        lax.fori_loop(0, nsup, super_body, 0)

        @pl.when(wid < xtra)
        def _():
            xrow = _NW * cpw + wid
            pltpu.sync_copy(dst_hbm.at[xrow], idxbuf.at[cpw])
            pltpu.sync_copy(msg_hbm.at[pl.ds(xrow * _CH, _CH)], msg_x)
            pltpu.sync_copy(msg_x, acc.at[idxbuf.at[cpw]], add=True)

        plsc.subcore_barrier()
        pltpu.sync_copy(acc.at[pl.ds(sid * _ZR, _ZR)],
                        out_hbm.at[cid, pl.ds(sid * _ZR, _ZR)])

    return k(msg, dst2d)


def _tc_msg(eap, xjp, W1p, b1p, W2p, b2p, W3, b3p, Rp, Sp, blk_off, n_edges):
    def body(eap_ref, xjp_ref, w1p, b1r, w2p, b2r, w3, b3r, rp, sp,
             out_ref):
        h = jnp.dot(eap_ref[...], w1p[...], preferred_element_type=jnp.float32)
        h = _gelu_exact(h + b1r[...])
        h = jnp.dot(h, w2p[...], preferred_element_type=jnp.float32)
        h = _gelu_exact(h + b2r[...])
        w3v = w3[...]
        b3v = b3r[...]
        xjv = xjp_ref[...]
        rpv = rp[...]
        spv = sp[...]
        acc = jnp.zeros((h.shape[0], 128), jnp.float32)
        for k in range(8):
            wp_k = jnp.dot(h[:, 64 * k:64 * (k + 1)], w3v,
                           preferred_element_type=jnp.float32)
            wp_k = wp_k + b3v[:, 256 * k:256 * (k + 1)]
            xrep_k = jnp.dot(xjv, rpv[:, 256 * k:256 * (k + 1)],
                             preferred_element_type=jnp.float32)
            acc = acc + jnp.dot(xrep_k * wp_k,
                                spv[256 * k:256 * (k + 1), :],
                                preferred_element_type=jnp.float32)
        out_ref[...] = acc

    return pl.pallas_call(
        body,
        grid=(n_edges // _TE,),
        in_specs=[
            pl.BlockSpec((_TR, 128), lambda i: (i + blk_off, 0)),
            pl.BlockSpec((_TR, 128), lambda i: (i, 0)),
            pl.BlockSpec((128, 8 * _KD), lambda i: (0, 0)),
            pl.BlockSpec((1, 8 * _KD), lambda i: (0, 0)),
            pl.BlockSpec((8 * _KD, 8 * _KD), lambda i: (0, 0)),
            pl.BlockSpec((1, 8 * _KD), lambda i: (0, 0)),
            pl.BlockSpec((_KD, _D * _D), lambda i: (0, 0)),
            pl.BlockSpec((1, 8 * _D * _D), lambda i: (0, 0)),
            pl.BlockSpec((128, 8 * _D * _D), lambda i: (0, 0)),
            pl.BlockSpec((8 * _D * _D, 128), lambda i: (0, 0)),
        ],
        out_specs=pl.BlockSpec((_TR, 128), lambda i: (i, 0)),
        out_shape=jax.ShapeDtypeStruct((n_edges * _D // 128, 128),
                                       jnp.float32),
    )(eap, xjp, W1p, b1p, W2p, b2p, W3, b3p, Rp, Sp)


def _tc_final(x, parts_a, parts_b, root, bias, use_gelu):
    def body(x_ref, pa_ref, pb_ref, r_ref, b_ref, out_ref):
        out = pa_ref[0] + pa_ref[1] + pb_ref[0] + pb_ref[1] + b_ref[...]
        out = out + jnp.dot(x_ref[...], r_ref[...],
                            preferred_element_type=jnp.float32)
        if use_gelu:
            out = _gelu_exact(out)
        out_ref[...] = out

    return pl.pallas_call(
        body,
        out_shape=jax.ShapeDtypeStruct((_N, _D), jnp.float32),
    )(x, parts_a, parts_b, root, bias)


def kernel(nodes, edge_index, edge_attr, W1, b1, W2, b2, W3, b3,
           root1, bias1, root2, bias2):
    src2d = edge_index[0].astype(jnp.int32).reshape(_E // _CH, _CH)
    dst2d = edge_index[1].astype(jnp.int32).reshape(_E // _CH, _CH)
    src_h = (src2d[:_NCKH], src2d[_NCKH:])
    dst_h = (dst2d[:_NCKH], dst2d[_NCKH:])
    eye8 = jnp.eye(8, dtype=jnp.float32)
    W1p = jnp.kron(eye8, W1)
    W2p = jnp.kron(eye8, W2)
    b1p = jnp.tile(b1, 8).reshape(1, 8 * _KD)
    b2p = jnp.tile(b2, 8).reshape(1, 8 * _KD)
    b3p = jnp.tile(b3, 8).reshape(1, 8 * _D * _D)
    la = jnp.arange(128, dtype=jnp.int32)[:, None]
    lb = jnp.arange(8 * _D * _D, dtype=jnp.int32)[None, :]
    Rp = (la == _D * (lb // (_D * _D))
          + (lb % (_D * _D)) // _D).astype(jnp.float32)
    lc = jnp.arange(8 * _D * _D, dtype=jnp.int32)[:, None]
    ld = jnp.arange(128, dtype=jnp.int32)[None, :]
    Sp = ((lc // (_D * _D) == ld // _D)
          & (lc % _D == ld % _D)).astype(jnp.float32)
    eap = edge_attr.reshape(_E * _ED // 128, 128)
    bias1r = bias1.reshape(1, _D)
    bias2r = bias2.reshape(1, _D)
    hblk = _EH * _D // 128 // _TR

    def gno_pass(x, root, biasr, use_gelu):
        xj = [_sc_gather(x, src_h[i], _NCKH) for i in range(2)]
        msg = [_tc_msg(eap, xj[i].reshape(_EH * _D // 128, 128),
                       W1p, b1p, W2p, b2p, W3, b3p, Rp, Sp,
                       i * hblk, _EH) for i in range(2)]
        parts = [_sc_scatter(msg[i].reshape(_EH, _D), dst_h[i], _NCKH)
                 for i in range(2)]
        return _tc_final(x, parts[0], parts[1], root, biasr, use_gelu)

    h = gno_pass(nodes, root1, bias1r, True)
    return gno_pass(h, root2, bias2r, False)


# speedup vs baseline: 7.2831x; 1.0288x over previous
import functools

import jax
import jax.numpy as jnp
from jax import lax
from jax.experimental import pallas as pl
from jax.experimental.pallas import tpu as pltpu
from jax.experimental.pallas import tpu_sc as plsc

_N = 10000
_E = 320000
_D = 16
_ED = 16
_KD = 64

_NC = 2
_NS = 16
_NW = _NC * _NS
_CH = 128
_ZR = _N // _NS

_EH = _E // 2
_NCKH = _EH // _CH
_TE = 16000
_TR = _TE // 8

_INV_SQRT2 = 0.7071067811865476


def _gelu_exact(x):
    return 0.5 * x * (1.0 + lax.erf(x * _INV_SQRT2))


def _sc_gather(table, idx2d, ncks):
    cpw = ncks // _NW
    xtra = ncks - cpw * _NW
    sup = 6 if cpw % 6 == 0 else 3
    nsup = cpw // sup
    srows = sup * _CH
    mesh = plsc.VectorSubcoreMesh(core_axis_name="c", subcore_axis_name="s")

    @functools.partial(
        pl.kernel,
        mesh=mesh,
        out_type=jax.ShapeDtypeStruct((ncks * _CH, _D), jnp.float32),
        scratch_types=[
            pltpu.VMEM((cpw + 1, _CH), jnp.int32),
            pltpu.VMEM((2, srows, _D), jnp.float32),
            pltpu.VMEM((_CH, _D), jnp.float32),
            pltpu.SemaphoreType.DMA,
            pltpu.SemaphoreType.DMA,
        ],
        compiler_params=pltpu.CompilerParams(use_tc_tiling_on_sc=False),
    )
    def k(table_hbm, idx_hbm, out_hbm, idxbuf, rows2, rows_x, gsem, wsem):
        wid = lax.axis_index("s") * _NC + lax.axis_index("c")
        row0 = wid * cpw
        pltpu.sync_copy(idx_hbm.at[pl.ds(row0, cpw)],
                        idxbuf.at[pl.ds(0, cpw)])

        def super_body(j, carry):
            p = lax.rem(j, 2)

            @pl.when(j >= 2)
            def _():
                pltpu.make_async_copy(
                    out_hbm.at[pl.ds(0, srows)], rows2.at[p], wsem).wait()

            handles = []
            for b in range(sup):
                handles.append(pltpu.async_copy(
                    table_hbm.at[idxbuf.at[j * sup + b]],
                    rows2.at[p, pl.ds(b * _CH, _CH)], gsem))
            for h in handles:
                h.wait()
            pltpu.async_copy(
                rows2.at[p],
                out_hbm.at[pl.ds((row0 + j * sup) * _CH, srows)], wsem)
            return carry

        lax.fori_loop(0, nsup, super_body, 0)
        pltpu.make_async_copy(
            out_hbm.at[pl.ds(0, srows)], rows2.at[0], wsem).wait()
        pltpu.make_async_copy(
            out_hbm.at[pl.ds(0, srows)], rows2.at[1], wsem).wait()

        @pl.when(wid < xtra)
        def _():
            xrow = _NW * cpw + wid
            pltpu.sync_copy(idx_hbm.at[xrow], idxbuf.at[cpw])
            pltpu.async_copy(
                table_hbm.at[idxbuf.at[cpw]], rows_x, gsem).wait()
            pltpu.sync_copy(rows_x, out_hbm.at[pl.ds(xrow * _CH, _CH)])

    return k(table, idx2d)


def _sc_scatter(msg, dst2d, ncks):
    cpw = ncks // _NW
    xtra = ncks - cpw * _NW
    sup = 6 if cpw % 6 == 0 else 3
    nsup = cpw // sup
    srows = sup * _CH
    mesh = plsc.VectorSubcoreMesh(core_axis_name="c", subcore_axis_name="s")

    @functools.partial(
        pl.kernel,
        mesh=mesh,
        out_type=jax.ShapeDtypeStruct((_NC, _N, _D), jnp.float32),
        scratch_types=[
            pltpu.VMEM((cpw + 1, _CH), jnp.int32),
            pltpu.VMEM((2, srows, _D), jnp.float32),
            pltpu.VMEM((_CH, _D), jnp.float32),
            pltpu.VMEM((_ZR, _D), jnp.float32),
            pltpu.VMEM_SHARED((_N, _D), jnp.float32),
            pltpu.SemaphoreType.DMA,
        ],
        compiler_params=pltpu.CompilerParams(use_tc_tiling_on_sc=False),
    )
    def k(msg_hbm, dst_hbm, out_hbm, idxbuf, msg2, msg_x, z_v, acc, lsem):
        cid = lax.axis_index("c")
        sid = lax.axis_index("s")
        wid = sid * _NC + cid
        row0 = wid * cpw
        zero = jnp.zeros((_D,), jnp.float32)

        def zb(i, carry):
            z_v[i, :] = zero
            return carry

        lax.fori_loop(0, _ZR, zb, 0)
        pltpu.sync_copy(z_v, acc.at[pl.ds(sid * _ZR, _ZR)])
        pltpu.sync_copy(dst_hbm.at[pl.ds(row0, cpw)],
                        idxbuf.at[pl.ds(0, cpw)])
        plsc.subcore_barrier()

        pltpu.async_copy(msg_hbm.at[pl.ds(row0 * _CH, srows)],
                         msg2.at[0], lsem)

        def super_body(j, carry):
            p = lax.rem(j, 2)
            pltpu.make_async_copy(
                msg_hbm.at[pl.ds(0, srows)], msg2.at[p], lsem).wait()

            @pl.when(j < nsup - 1)
            def _():
                pltpu.async_copy(
                    msg_hbm.at[pl.ds((row0 + (j + 1) * sup) * _CH, srows)],
                    msg2.at[1 - p], lsem)

            for b in range(sup):
                pltpu.sync_copy(msg2.at[p, pl.ds(b * _CH, _CH)],
                                acc.at[idxbuf.at[j * sup + b]], add=True)
            return carry

        lax.fori_loop(0, nsup, super_body, 0)

        @pl.when(wid < xtra)
        def _():
            xrow = _NW * cpw + wid
            pltpu.sync_copy(dst_hbm.at[xrow], idxbuf.at[cpw])
            pltpu.sync_copy(msg_hbm.at[pl.ds(xrow * _CH, _CH)], msg_x)
            pltpu.sync_copy(msg_x, acc.at[idxbuf.at[cpw]], add=True)

        plsc.subcore_barrier()
        pltpu.sync_copy(acc.at[pl.ds(sid * _ZR, _ZR)],
                        out_hbm.at[cid, pl.ds(sid * _ZR, _ZR)])

    return k(msg, dst2d)


def _tc_msg(eap, xjp, W1p, b1p, W2p, b2p, W3, b3p, Rp, Sp, blk_off, n_edges):
    def body(eap_ref, xjp_ref, w1p, b1r, w2p, b2r, w3, b3r, rp, sp,
             out_ref):
        h = jnp.dot(eap_ref[...], w1p[...], preferred_element_type=jnp.float32)
        h = _gelu_exact(h + b1r[...])
        h = jnp.dot(h, w2p[...], preferred_element_type=jnp.float32)
        h = _gelu_exact(h + b2r[...])
        w3v = w3[...]
        b3v = b3r[...]
        xjv = xjp_ref[...]
        rpv = rp[...]
        spv = sp[...]
        acc = jnp.zeros((h.shape[0], 128), jnp.float32)
        for k in range(8):
            wp_k = jnp.dot(h[:, 64 * k:64 * (k + 1)], w3v,
                           preferred_element_type=jnp.float32)
            wp_k = wp_k + b3v[:, 256 * k:256 * (k + 1)]
            xrep_k = jnp.dot(xjv, rpv[:, 256 * k:256 * (k + 1)],
                             preferred_element_type=jnp.float32)
            acc = acc + jnp.dot(xrep_k * wp_k,
                                spv[256 * k:256 * (k + 1), :],
                                preferred_element_type=jnp.float32)
        out_ref[...] = acc

    return pl.pallas_call(
        body,
        grid=(n_edges // _TE,),
        in_specs=[
            pl.BlockSpec((_TR, 128), lambda i: (i + blk_off, 0)),
            pl.BlockSpec((_TR, 128), lambda i: (i, 0)),
            pl.BlockSpec((128, 8 * _KD), lambda i: (0, 0)),
            pl.BlockSpec((1, 8 * _KD), lambda i: (0, 0)),
            pl.BlockSpec((8 * _KD, 8 * _KD), lambda i: (0, 0)),
            pl.BlockSpec((1, 8 * _KD), lambda i: (0, 0)),
            pl.BlockSpec((_KD, _D * _D), lambda i: (0, 0)),
            pl.BlockSpec((1, 8 * _D * _D), lambda i: (0, 0)),
            pl.BlockSpec((128, 8 * _D * _D), lambda i: (0, 0)),
            pl.BlockSpec((8 * _D * _D, 128), lambda i: (0, 0)),
        ],
        out_specs=pl.BlockSpec((_TR, 128), lambda i: (i, 0)),
        out_shape=jax.ShapeDtypeStruct((n_edges * _D // 128, 128),
                                       jnp.float32),
    )(eap, xjp, W1p, b1p, W2p, b2p, W3, b3p, Rp, Sp)


def _tc_final(x, parts_a, parts_b, root, bias, use_gelu):
    def body(x_ref, pa_ref, pb_ref, r_ref, b_ref, out_ref):
        out = pa_ref[0] + pa_ref[1] + pb_ref[0] + pb_ref[1] + b_ref[...]
        out = out + jnp.dot(x_ref[...], r_ref[...],
                            preferred_element_type=jnp.float32)
        if use_gelu:
            out = _gelu_exact(out)
        out_ref[...] = out

    return pl.pallas_call(
        body,
        out_shape=jax.ShapeDtypeStruct((_N, _D), jnp.float32),
    )(x, parts_a, parts_b, root, bias)


def kernel(nodes, edge_index, edge_attr, W1, b1, W2, b2, W3, b3,
           root1, bias1, root2, bias2):
    src2d = edge_index[0].astype(jnp.int32).reshape(_E // _CH, _CH)
    dst2d = edge_index[1].astype(jnp.int32).reshape(_E // _CH, _CH)
    src_h = (src2d[:_NCKH], src2d[_NCKH:])
    dst_h = (dst2d[:_NCKH], dst2d[_NCKH:])
    eye8 = jnp.eye(8, dtype=jnp.float32)
    W1p = jnp.kron(eye8, W1)
    W2p = jnp.kron(eye8, W2)
    b1p = jnp.tile(b1, 8).reshape(1, 8 * _KD)
    b2p = jnp.tile(b2, 8).reshape(1, 8 * _KD)
    b3p = jnp.tile(b3, 8).reshape(1, 8 * _D * _D)
    la = jnp.arange(128, dtype=jnp.int32)[:, None]
    lb = jnp.arange(8 * _D * _D, dtype=jnp.int32)[None, :]
    Rp = (la == _D * (lb // (_D * _D))
          + (lb % (_D * _D)) // _D).astype(jnp.float32)
    lc = jnp.arange(8 * _D * _D, dtype=jnp.int32)[:, None]
    ld = jnp.arange(128, dtype=jnp.int32)[None, :]
    Sp = ((lc // (_D * _D) == ld // _D)
          & (lc % _D == ld % _D)).astype(jnp.float32)
    eap = edge_attr.reshape(_E * _ED // 128, 128)
    bias1r = bias1.reshape(1, _D)
    bias2r = bias2.reshape(1, _D)
    hblk = _EH * _D // 128 // _TR

    def gno_pass(x, root, biasr, use_gelu):
        xj = [_sc_gather(x, src_h[i], _NCKH) for i in range(2)]
        msg = [_tc_msg(eap, xj[i].reshape(_EH * _D // 128, 128),
                       W1p, b1p, W2p, b2p, W3, b3p, Rp, Sp,
                       i * hblk, _EH) for i in range(2)]
        parts = [_sc_scatter(msg[i].reshape(_EH, _D), dst_h[i], _NCKH)
                 for i in range(2)]
        return _tc_final(x, parts[0], parts[1], root, biasr, use_gelu)

    h = gno_pass(nodes, root1, bias1r, True)
    return gno_pass(h, root2, bias2r, False)
